# baseline (speedup 1.0000x reference)
import jax
import jax.numpy as jnp
from jax import lax
from jax.experimental import pallas as pl
from jax.experimental.pallas import tpu as pltpu
from jax.experimental.pallas import tpu_sc as plsc

_NUM_CLASSES = 1000000
_HIDDEN = 64
_DROPOUT_PROB = 0.1

_NC = 2
_NS = 16
_NW = _NC * _NS
_B = 16384
_BPW = _B // _NW
_CHUNK = 128
_NCHUNK = _BPW // _CHUNK
_LANES = 16


def _embed_body(labels_hbm, drop_hbm, table_hbm, out_hbm, idx_v, drop_v, rows_v, sem):
    wid = lax.axis_index("s") * _NC + lax.axis_index("c")
    pltpu.sync_copy(labels_hbm.at[wid], idx_v)
    pltpu.sync_copy(drop_hbm.at[wid], drop_v)
    for j in range(_NCHUNK):
        for i in range(_CHUNK // _LANES):
            s = pl.ds(i * _LANES, _LANES)
            lab = idx_v[j, s]
            flag = drop_v[j, s]
            idx_v[j, s] = jnp.where(flag != 0, _NUM_CLASSES, lab)
    copies = [
        pltpu.async_copy(
            table_hbm.at[idx_v.at[j]],
            rows_v.at[pl.ds(j * _CHUNK, _CHUNK)],
            sem,
        )
        for j in range(_NCHUNK)
    ]
    for c in copies:
        c.wait()
    pltpu.sync_copy(rows_v, out_hbm.at[wid])


def kernel(labels, train, embedding_table):
    b = labels.shape[0]
    rand_drop = jax.random.uniform(jax.random.key(1), (b,)) < _DROPOUT_PROB
    use_dropout = jnp.logical_and(_DROPOUT_PROB > 0, train != 0)
    drop = jnp.logical_and(rand_drop, use_dropout).astype(jnp.int32)
    labels32 = labels.astype(jnp.int32).reshape(_NW, _NCHUNK, _CHUNK)
    drop = drop.reshape(_NW, _NCHUNK, _CHUNK)

    mesh = plsc.VectorSubcoreMesh(core_axis_name="c", subcore_axis_name="s")
    out = pl.kernel(
        _embed_body,
        out_type=jax.ShapeDtypeStruct((_NW, _BPW, _HIDDEN), jnp.float32),
        mesh=mesh,
        compiler_params=pltpu.CompilerParams(use_tc_tiling_on_sc=False),
        scratch_types=[
            pltpu.VMEM((_NCHUNK, _CHUNK), jnp.int32),
            pltpu.VMEM((_NCHUNK, _CHUNK), jnp.int32),
            pltpu.VMEM((_BPW, _HIDDEN), jnp.float32),
            pltpu.SemaphoreType.DMA,
        ],
    )(labels32, drop, embedding_table)
    return out.reshape(b, _HIDDEN)


# speedup vs baseline: 2.0497x; 2.0497x over previous
import jax
import jax.numpy as jnp
from jax import lax
from jax.experimental import pallas as pl
from jax.experimental.pallas import tpu as pltpu
from jax.experimental.pallas import tpu_sc as plsc

_NUM_CLASSES = 1000000
_V = _NUM_CLASSES + 1
_D = 64
_B = 16384
_DROPOUT_PROB = 0.1

_NWORK = 32
_LANES = 16
_WIN = 256
_TAIL_BASE = 999936
_TAIL_N = _V - _TAIL_BASE
_NWIN = _TAIL_BASE // _WIN
_CAP = 1024
_CHUNK = 4096
_Q, _R = divmod(_NWIN, _NWORK)
_CFG_OFF = (_NUM_CLASSES - _TAIL_BASE) * _D


def _body(labels_hbm, flags_hbm, tableT_hbm, tail_hbm, out_hbm,
          labc_v, flagc_v, mlab_v, mpos_v, mflag_v, tail_v,
          win0_v, win1_v, rows_v, sem0, sem1, semo):
    wid = lax.axis_index("s") * 2 + lax.axis_index("c")
    iota = lax.iota(jnp.int32, _LANES)

    nw = _Q + jnp.where(wid < _R, 1, 0).astype(jnp.int32)
    w0 = wid * _Q + jnp.minimum(wid, _R)
    lo = w0 * _WIN
    hi = jnp.where(wid == _NWORK - 1, jnp.int32(2**30), (w0 + nw) * _WIN)

    pltpu.sync_copy(tail_hbm, tail_v)

    def scan_chunk(c, cnt):
        pltpu.sync_copy(labels_hbm.at[pl.ds(c * _CHUNK, _CHUNK)], labc_v)
        pltpu.sync_copy(flags_hbm.at[pl.ds(c * _CHUNK, _CHUNK)], flagc_v)

        def g_body(g, cnt):
            lab = labc_v[pl.ds(g * _LANES, _LANES)]
            fl = flagc_v[pl.ds(g * _LANES, _LANES)]
            m = jnp.logical_and(lab >= lo, lab < hi)
            plsc.store_compressed(mlab_v.at[pl.ds(cnt, _LANES)], lab, mask=m)
            pos = iota + (c * _CHUNK + g * _LANES)
            plsc.store_compressed(mpos_v.at[pl.ds(cnt, _LANES)], pos, mask=m)
            plsc.store_compressed(mflag_v.at[pl.ds(cnt, _LANES)], fl, mask=m)
            return cnt + plsc.all_reduce_population_count(m)[0]

        return lax.fori_loop(0, _CHUNK // _LANES, g_body, cnt)

    cnt = lax.fori_loop(0, _B // _CHUNK, scan_chunk, jnp.int32(0))
    jmax = (cnt + _LANES - 1) // _LANES

    def emit_row(slot, pos):
        pltpu.async_copy(
            rows_v.at[pl.ds(slot * _D, _D)],
            out_hbm.at[pl.ds(pos * _D, _D)],
            semo,
        )

    def rescan(wb, win):
        def jbody(j, carry):
            valid = (iota + j * _LANES) < cnt
            ml = mlab_v[pl.ds(j * _LANES, _LANES)]
            mf = mflag_v[pl.ds(j * _LANES, _LANES)]
            mp = mpos_v[pl.ds(j * _LANES, _LANES)]
            inw = valid & (ml >= wb) & (ml < wb + _WIN)
            pc = plsc.all_reduce_population_count(inw)[0]

            @pl.when(pc > 0)
            def _():
                inw32 = inw.astype(jnp.int32)
                for k in range(_LANES):
                    @pl.when(inw32[k] > 0)
                    def _():
                        slot = j * _LANES + k
                        lab = ml[k]
                        fl = mf[k]
                        pos = mp[k]

                        @pl.when(fl == 0)
                        def _():
                            colv = jnp.zeros((_LANES,), jnp.int32) + (lab - wb)
                            for g in range(_D // _LANES):
                                rows_v[pl.ds(slot * _D + g * _LANES, _LANES)] = (
                                    plsc.load_gather(
                                        win, [iota + g * _LANES, colv]
                                    )
                                )

                        @pl.when(fl != 0)
                        def _():
                            for g in range(_D // _LANES):
                                rows_v[pl.ds(slot * _D + g * _LANES, _LANES)] = (
                                    tail_v[pl.ds(_CFG_OFF + g * _LANES, _LANES)]
                                )

                        emit_row(slot, pos)

            return carry

        lax.fori_loop(0, jmax, jbody, 0)

    def fire(widx, win, sem):
        wb_lanes = pl.multiple_of((w0 + widx) * _WIN, _WIN)
        pltpu.async_copy(tableT_hbm.at[:, pl.ds(wb_lanes, _WIN)], win, sem)

    def wait_win(widx, win, sem):
        wb_lanes = pl.multiple_of((w0 + widx) * _WIN, _WIN)
        pltpu.make_async_copy(
            tableT_hbm.at[:, pl.ds(wb_lanes, _WIN)], win, sem
        ).wait()

    @pl.when(nw > 0)
    def _():
        fire(0, win0_v, sem0)

    @pl.when(nw > 1)
    def _():
        fire(1, win1_v, sem1)

    def wpair(i, carry):
        for b in range(2):
            win = win0_v if b == 0 else win1_v
            sem = sem0 if b == 0 else sem1
            widx = i * 2 + b

            @pl.when(widx < nw)
            def _():
                wait_win(widx, win, sem)
                rescan((w0 + widx) * _WIN, win)

                @pl.when(widx + 2 < nw)
                def _():
                    fire(widx + 2, win, sem)

        return carry

    lax.fori_loop(0, (nw + 1) // 2, wpair, 0)

    def tbody(j, carry):
        valid = (iota + j * _LANES) < cnt
        ml = mlab_v[pl.ds(j * _LANES, _LANES)]
        mf = mflag_v[pl.ds(j * _LANES, _LANES)]
        mp = mpos_v[pl.ds(j * _LANES, _LANES)]
        int_ = valid & (ml >= _TAIL_BASE)
        pc = plsc.all_reduce_population_count(int_)[0]

        @pl.when(pc > 0)
        def _():
            int32_ = int_.astype(jnp.int32)
            for k in range(_LANES):
                @pl.when(int32_[k] > 0)
                def _():
                    slot = j * _LANES + k
                    lab = ml[k]
                    fl = mf[k]
                    pos = mp[k]
                    src = jnp.where(fl != 0, _NUM_CLASSES, lab) - _TAIL_BASE
                    for g in range(_D // _LANES):
                        rows_v[pl.ds(slot * _D + g * _LANES, _LANES)] = (
                            tail_v[pl.ds(src * _D + g * _LANES, _LANES)]
                        )
                    emit_row(slot, pos)

        return carry

    lax.fori_loop(0, jmax, tbody, 0)

    def dbody(i, carry):
        pltpu.make_async_copy(
            tableT_hbm.at[0, pl.ds(0, _D)], rows_v.at[pl.ds(0, _D)], semo
        ).wait()
        return carry

    lax.fori_loop(0, cnt, dbody, 0)


def kernel(labels, train, embedding_table):
    b = labels.shape[0]
    rand_drop = jax.random.uniform(jax.random.key(1), (b,)) < _DROPOUT_PROB
    use_dropout = jnp.logical_and(_DROPOUT_PROB > 0, train != 0)
    flags = jnp.logical_and(rand_drop, use_dropout).astype(jnp.int32)
    labels32 = labels.astype(jnp.int32)
    tableT = embedding_table.T
    tail = embedding_table[_TAIL_BASE:].reshape(-1)

    mesh = plsc.VectorSubcoreMesh(core_axis_name="c", subcore_axis_name="s")
    out = pl.kernel(
        _body,
        out_type=jax.ShapeDtypeStruct((_B * _D,), jnp.float32),
        mesh=mesh,
        compiler_params=pltpu.CompilerParams(needs_layout_passes=False),
        scratch_types=[
            pltpu.VMEM((_CHUNK,), jnp.int32),
            pltpu.VMEM((_CHUNK,), jnp.int32),
            pltpu.VMEM((_CAP + _LANES,), jnp.int32),
            pltpu.VMEM((_CAP + _LANES,), jnp.int32),
            pltpu.VMEM((_CAP + _LANES,), jnp.int32),
            pltpu.VMEM((_TAIL_N * _D,), jnp.float32),
            pltpu.VMEM((_D, _WIN), jnp.float32),
            pltpu.VMEM((_D, _WIN), jnp.float32),
            pltpu.VMEM((_CAP * _D,), jnp.float32),
            pltpu.SemaphoreType.DMA,
            pltpu.SemaphoreType.DMA,
            pltpu.SemaphoreType.DMA,
        ],
    )(labels32, flags, tableT, tail)
    return out.reshape(b, _D)
